# baseline (speedup 1.0000x reference)
import functools

import jax
import jax.numpy as jnp
import numpy as np
from jax import lax
from jax.experimental import pallas as pl
from jax.experimental.pallas import tpu as pltpu
from jax.experimental.pallas import tpu_sc as plsc

_CLIP = np.array([(1, 0, 1), (2, 0, 1), (3, 0, -1), (4, 0, -1), (5, 0, -1),
                  (6, 0, -1), (7, 0, -1), (8, 0, -1), (9, 0, -1), (12, 0, -1),
                  (13, 1, 1), (14, 1, -1), (16, 1, 1), (17, 1, -1),
                  (18, 1, 1), (19, 1, -1)], dtype=np.int64)
_ZERO = np.array([(10, 0), (10, 1), (10, 2), (11, 0), (11, 1), (11, 2),
                  (15, 0), (15, 1), (15, 2), (20, 1), (21, 1)], dtype=np.int64)

_N_ROWS = 16384
_N_COLS = 63
_N_TERMS = len(_CLIP) + len(_ZERO)
_SCALE = 1.0 / (_N_ROWS * _N_TERMS)

_P_COLS = tuple(int((j - 1) * 3 + a) for j, a, s in _CLIP if s > 0)
_N_COLS_NEG = tuple(int((j - 1) * 3 + a) for j, a, s in _CLIP if s < 0)
_Z_COLS = tuple(int((j - 1) * 3 + a) for j, a in _ZERO)

_NW = 32
_ROWS_PER_W = _N_ROWS // _NW
_BLOCKS = _ROWS_PER_W // 16

_mesh = plsc.VectorSubcoreMesh(core_axis_name="c", subcore_axis_name="s")


@functools.partial(
    pl.kernel,
    out_type=jax.ShapeDtypeStruct((_NW * 16,), jnp.float32),
    mesh=_mesh,
    scratch_types=[
        pltpu.VMEM((_ROWS_PER_W, _N_COLS), jnp.float32),
        pltpu.VMEM((16,), jnp.float32),
    ],
    compiler_params=pltpu.CompilerParams(needs_layout_passes=False),
)
def _sc_partial_sums(pose_hbm, out_hbm, x_v, acc_v):
    wid = lax.axis_index("s") * 2 + lax.axis_index("c")
    row0 = wid * _ROWS_PER_W
    with jax.named_scope("dma_in"):
        pltpu.sync_copy(pose_hbm.at[pl.ds(row0, _ROWS_PER_W)], x_v)

    lane = lax.iota(jnp.int32, 16)
    zero = jnp.zeros((16,), jnp.float32)

    def body(b, accs):
        p0, p1, n0, n1, z0, z1 = accs
        rows = lane + b * 16
        pr = []
        for c in _P_COLS:
            x = plsc.load_gather(x_v, [rows, jnp.full((16,), c, jnp.int32)])
            pr.append(jnp.maximum(x, 0.0))
        nr = []
        for c in _N_COLS_NEG:
            x = plsc.load_gather(x_v, [rows, jnp.full((16,), c, jnp.int32)])
            nr.append(jnp.minimum(x, 0.0))
        zr = []
        for c in _Z_COLS:
            x = plsc.load_gather(x_v, [rows, jnp.full((16,), c, jnp.int32)])
            zr.append(jnp.abs(x))
        p0 = p0 + sum(pr[0::2], zero)
        p1 = p1 + sum(pr[1::2], zero)
        n0 = n0 + sum(nr[0::2], zero)
        n1 = n1 + sum(nr[1::2], zero)
        z0 = z0 + sum(zr[0::2], zero)
        z1 = z1 + sum(zr[1::2], zero)
        return (p0, p1, n0, n1, z0, z1)

    init = (zero,) * 6
    with jax.named_scope("reduce"):
        p0, p1, n0, n1, z0, z1 = lax.fori_loop(0, _BLOCKS, body, init)
    acc = ((p0 + p1) - (n0 + n1) + (z0 + z1)) * jnp.float32(_SCALE)
    acc_v[...] = acc
    pltpu.sync_copy(acc_v, out_hbm.at[pl.ds(wid * 16, 16)])


def kernel(pose):
    partials = _sc_partial_sums(pose)
    return jnp.sum(partials)


# speedup vs baseline: 1.4510x; 1.4510x over previous
import functools

import jax
import jax.numpy as jnp
import numpy as np
from jax import lax
from jax.experimental import pallas as pl
from jax.experimental.pallas import tpu as pltpu
from jax.experimental.pallas import tpu_sc as plsc

_CLIP = np.array([(1, 0, 1), (2, 0, 1), (3, 0, -1), (4, 0, -1), (5, 0, -1),
                  (6, 0, -1), (7, 0, -1), (8, 0, -1), (9, 0, -1), (12, 0, -1),
                  (13, 1, 1), (14, 1, -1), (16, 1, 1), (17, 1, -1),
                  (18, 1, 1), (19, 1, -1)], dtype=np.int64)
_ZERO = np.array([(10, 0), (10, 1), (10, 2), (11, 0), (11, 1), (11, 2),
                  (15, 0), (15, 1), (15, 2), (20, 1), (21, 1)], dtype=np.int64)

_N_ROWS = 16384
_N_COLS = 63
_N_TERMS = len(_CLIP) + len(_ZERO)
_SCALE = 1.0 / (_N_ROWS * _N_TERMS)

_P_COLS = tuple(int((j - 1) * 3 + a) for j, a, s in _CLIP if s > 0)
_N_COLS_NEG = tuple(int((j - 1) * 3 + a) for j, a, s in _CLIP if s < 0)
_Z_COLS = tuple(int((j - 1) * 3 + a) for j, a in _ZERO)
_USED = ([(c, 0) for c in _P_COLS] + [(c, 1) for c in _N_COLS_NEG]
         + [(c, 2) for c in _Z_COLS])

_NW = 32
_COLS_PER_W = _N_ROWS // _NW
_VECS = _COLS_PER_W // 16

_mesh = plsc.VectorSubcoreMesh(core_axis_name="c", subcore_axis_name="s")


@functools.partial(
    pl.kernel,
    out_type=jax.ShapeDtypeStruct((_NW * 16,), jnp.float32),
    mesh=_mesh,
    scratch_types=[
        pltpu.VMEM((_N_TERMS * _COLS_PER_W,), jnp.float32),
        pltpu.VMEM((16,), jnp.float32),
        pltpu.SemaphoreType.DMA,
    ],
)
def _sc_partial_sums(pose_t_hbm, out_hbm, x_v, acc_v, sem):
    wid = lax.axis_index("s") * 2 + lax.axis_index("c")
    col0 = wid * _COLS_PER_W

    handles = []
    for k, (c, _) in enumerate(_USED):
        handles.append(pltpu.async_copy(
            pose_t_hbm.at[c, pl.ds(col0, _COLS_PER_W)],
            x_v.at[pl.ds(k * _COLS_PER_W, _COLS_PER_W)],
            sem,
        ))
    for h in handles:
        h.wait()

    zero = jnp.zeros((16,), jnp.float32)

    def body(j, accs):
        p0, p1, n0, n1, z0, z1 = accs
        off = j * 16
        res = [[], [], []]
        for k, (_, kind) in enumerate(_USED):
            x = x_v[pl.ds(k * _COLS_PER_W + off, 16)]
            if kind == 0:
                res[0].append(jnp.maximum(x, 0.0))
            elif kind == 1:
                res[1].append(jnp.minimum(x, 0.0))
            else:
                res[2].append(jnp.abs(x))
        p0 = p0 + sum(res[0][0::2], zero)
        p1 = p1 + sum(res[0][1::2], zero)
        n0 = n0 + sum(res[1][0::2], zero)
        n1 = n1 + sum(res[1][1::2], zero)
        z0 = z0 + sum(res[2][0::2], zero)
        z1 = z1 + sum(res[2][1::2], zero)
        return (p0, p1, n0, n1, z0, z1)

    init = (zero,) * 6
    p0, p1, n0, n1, z0, z1 = lax.fori_loop(0, _VECS, body, init)
    acc = ((p0 + p1) - (n0 + n1) + (z0 + z1)) * jnp.float32(_SCALE)
    acc_v[...] = acc
    pltpu.sync_copy(acc_v, out_hbm.at[pl.ds(wid * 16, 16)])


def kernel(pose):
    partials = _sc_partial_sums(pose.T)
    return jnp.sum(partials)
